# Initial kernel scaffold; baseline (speedup 1.0000x reference)
import jax
import jax.numpy as jnp
from jax.experimental import pallas as pl


def kernel(seq1, seq2, lbl, adjs, sparse, msk, samp_bias1, samp_bias2, W0, b0, a0, Wb0, bb0, W1, b1, a1, Wb1, bb1):
    raise NotImplementedError("write your pallas kernel here")


# speedup vs baseline: 3.0889x; 3.0889x over previous
import functools

import jax
import jax.numpy as jnp
from jax import lax
from jax.experimental import pallas as pl
from jax.experimental.pallas import tpu as pltpu
from jax.experimental.pallas import tpu_sc as plsc

N = 10000
E = 320000
F = 128
H = 128
P = 2

NT = 16
CH = 128
EPT = E // NT
G = 16
NG = -(-EPT // (G * CH))
EPT_PAD = NG * G * CH
PAD = EPT_PAD - EPT
ZR = 64
ZPT = 632
ACC_ROWS = NT * ZPT
XPT = 632
XPT_LAST = N - (NT - 1) * XPT

NB = 1000
NBLK = N // NB


def _mm_body(seq_ref, w_ref, out_ref):
    out_ref[...] = jnp.dot(seq_ref[0], w_ref[...],
                           preferred_element_type=jnp.float32)


def _matmul(seq, w):
    return pl.pallas_call(
        _mm_body,
        grid=(NBLK,),
        in_specs=[
            pl.BlockSpec((1, NB, F), lambda b: (0, b, 0)),
            pl.BlockSpec((F, H), lambda b: (0, 0)),
        ],
        out_specs=pl.BlockSpec((NB, H), lambda b: (b, 0)),
        out_shape=jax.ShapeDtypeStruct((N, H), jnp.float32),
    )(seq, w)


def _segsum_body(x0, x1, x2, x3, src_hbm, dst_hbm, out_hbm,
                 src_v, dst_v, rows_v, zero_v, acc_sh, sem):
    core = lax.axis_index("c")
    tid = lax.axis_index("s")
    xs = [x0, x1, x2, x3]

    def zrow(r, carry):
        for g in range(H // 16):
            zero_v[r, pl.ds(g * 16, 16)] = jnp.zeros((16,), jnp.float32)
        return carry
    lax.fori_loop(0, ZR, zrow, 0)

    for v in range(P):
        @pl.when(core == v)
        def _view():
            for rep in range(2):
                combo = rep * P + v
                x_hbm = xs[combo]
                zbase = pl.multiple_of(tid * ZPT, 8)
                for q in range(ZPT // ZR):
                    pltpu.sync_copy(zero_v,
                                    acc_sh.at[pl.ds(zbase + q * ZR, ZR)])
                rem = ZPT % ZR
                if rem:
                    pltpu.sync_copy(zero_v.at[pl.ds(0, rem)],
                                    acc_sh.at[pl.ds(zbase + (ZPT // ZR) * ZR, rem)])
                plsc.subcore_barrier()

                def group(g, carry):
                    pltpu.sync_copy(src_hbm.at[v, tid, g], src_v)
                    pltpu.sync_copy(dst_hbm.at[v, tid, g], dst_v)
                    for k in range(G):
                        pltpu.async_copy(x_hbm.at[src_v.at[k]], rows_v, sem).wait()
                        pltpu.sync_copy(rows_v, acc_sh.at[dst_v.at[k]], add=True)
                    return carry
                lax.fori_loop(0, NG, group, 0)
                plsc.subcore_barrier()
                eb = pl.multiple_of(tid * XPT, 8)

                @pl.when(tid < NT - 1)
                def _exp():
                    pltpu.sync_copy(acc_sh.at[pl.ds(eb, XPT)],
                                    out_hbm.at[combo, pl.ds(eb, XPT)])

                @pl.when(tid == NT - 1)
                def _exp_last():
                    pltpu.sync_copy(acc_sh.at[pl.ds((NT - 1) * XPT, XPT_LAST)],
                                    out_hbm.at[combo, pl.ds((NT - 1) * XPT, XPT_LAST)])
                plsc.subcore_barrier()


@functools.cache
def _make_segsum():
    return pl.kernel(
        _segsum_body,
        out_type=jax.ShapeDtypeStruct((4, N, H), jnp.float32),
        mesh=plsc.VectorSubcoreMesh(core_axis_name="c", subcore_axis_name="s",
                                    num_cores=P, num_subcores=NT),
        scratch_types=[
            pltpu.VMEM((G, CH), jnp.int32),
            pltpu.VMEM((G, CH), jnp.int32),
            pltpu.VMEM((CH, H), jnp.float32),
            pltpu.VMEM((ZR, H), jnp.float32),
            pltpu.VMEM_SHARED((ACC_ROWS, H), jnp.float32),
            pltpu.SemaphoreType.DMA,
        ],
    )


def _segsum(*args):
    return _make_segsum()(*args)


def _act_body(agg1_ref, agg2_ref, b_ref, a_ref, msk_ref,
              hh_ref, h2_ref, colsum_ref, msksum_ref):
    v = pl.program_id(0)
    b = pl.program_id(1)
    bias = b_ref[0, 0]
    alpha = a_ref[0, 0, 0]
    g1 = agg1_ref[0] + bias[None, :]
    h1 = jnp.where(g1 > 0, g1, alpha * g1)
    g2 = agg2_ref[0] + bias[None, :]
    h2 = jnp.where(g2 > 0, g2, alpha * g2)
    hh_ref[0, 0] = h1
    h2_ref[0] = h2
    m = msk_ref[0, 0]
    part = jnp.sum(h1 * m[:, None], axis=0)

    @pl.when(b == 0)
    def _init():
        colsum_ref[...] = part[None, None, :]

    @pl.when(b != 0)
    def _acc():
        colsum_ref[...] = colsum_ref[...] + part[None, None, :]

    @pl.when(jnp.logical_and(v == 0, b == 0))
    def _minit():
        msksum_ref[...] = jnp.sum(m).reshape(1, 1)

    @pl.when(jnp.logical_and(v == 0, b != 0))
    def _macc():
        msksum_ref[...] = msksum_ref[...] + jnp.sum(m).reshape(1, 1)


def _activate(aggs, bs, alphas, msk):
    return pl.pallas_call(
        _act_body,
        grid=(P, NBLK),
        in_specs=[
            pl.BlockSpec((1, NB, H), lambda v, b: (v, b, 0)),
            pl.BlockSpec((1, NB, H), lambda v, b: (v + P, b, 0)),
            pl.BlockSpec((1, 1, H), lambda v, b: (v, 0, 0)),
            pl.BlockSpec((1, 1, 1), lambda v, b: (v, 0, 0)),
            pl.BlockSpec((1, 1, NB), lambda v, b: (b, 0, 0)),
        ],
        out_specs=[
            pl.BlockSpec((1, 1, NB, H), lambda v, b: (v, 0, b, 0)),
            pl.BlockSpec((1, NB, H), lambda v, b: (v, b, 0)),
            pl.BlockSpec((1, 1, H), lambda v, b: (v, 0, 0)),
            pl.BlockSpec((1, 1), lambda v, b: (0, 0)),
        ],
        out_shape=[
            jax.ShapeDtypeStruct((P, 1, N, H), jnp.float32),
            jax.ShapeDtypeStruct((P, N, H), jnp.float32),
            jax.ShapeDtypeStruct((P, 1, H), jnp.float32),
            jax.ShapeDtypeStruct((1, 1), jnp.float32),
        ],
    )(aggs, aggs, bs, alphas, msk)


def _bce(x, y):
    return jnp.maximum(x, 0.0) - x * y + jnp.log(1.0 + jnp.exp(-jnp.abs(x)))


def _loss_body(hh_ref, h2_ref, colsum_ref, msksum_ref, wb_ref, bb_ref,
               lbl1_ref, lbl2_ref, sb1_ref, sb2_ref, loss_ref):
    v = pl.program_id(0)
    b = pl.program_id(1)
    c = colsum_ref[0, 0] / msksum_ref[0, 0]
    c = 1.0 / (1.0 + jnp.exp(-c))
    u = jnp.sum(wb_ref[0] * c[None, :], axis=1)
    bb = bb_ref[0, 0, 0]
    sc1 = jnp.sum(hh_ref[0, 0] * u[None, :], axis=1) + bb + sb1_ref[0, 0]
    sc2 = jnp.sum(h2_ref[0] * u[None, :], axis=1) + bb + sb2_ref[0, 0]
    part = (jnp.sum(_bce(sc1, lbl1_ref[0, 0]))
            + jnp.sum(_bce(sc2, lbl2_ref[0, 0])))

    @pl.when(jnp.logical_and(v == 0, b == 0))
    def _init():
        loss_ref[...] = jnp.zeros((1, 1), jnp.float32)

    loss_ref[...] = loss_ref[...] + part.reshape(1, 1)

    @pl.when(jnp.logical_and(v == P - 1, b == NBLK - 1))
    def _fin():
        loss_ref[...] = loss_ref[...] * (1.0 / (2.0 * N * P))


def _loss(hh, h2, colsum, msksum, wbs, bbs, lbl, sb1, sb2):
    return pl.pallas_call(
        _loss_body,
        grid=(P, NBLK),
        in_specs=[
            pl.BlockSpec((1, 1, NB, H), lambda v, b: (v, 0, b, 0)),
            pl.BlockSpec((1, NB, H), lambda v, b: (v, b, 0)),
            pl.BlockSpec((1, 1, H), lambda v, b: (v, 0, 0)),
            pl.BlockSpec((1, 1), lambda v, b: (0, 0)),
            pl.BlockSpec((1, H, H), lambda v, b: (v, 0, 0)),
            pl.BlockSpec((1, 1, 1), lambda v, b: (v, 0, 0)),
            pl.BlockSpec((1, 1, NB), lambda v, b: (b, 0, 0)),
            pl.BlockSpec((1, 1, NB), lambda v, b: (NBLK + b, 0, 0)),
            pl.BlockSpec((1, 1, NB), lambda v, b: (b, 0, 0)),
            pl.BlockSpec((1, 1, NB), lambda v, b: (b, 0, 0)),
        ],
        out_specs=pl.BlockSpec((1, 1), lambda v, b: (0, 0)),
        out_shape=jax.ShapeDtypeStruct((1, 1), jnp.float32),
    )(hh, h2, colsum, msksum, wbs, bbs, lbl, lbl, sb1, sb2)


def kernel(seq1, seq2, lbl, adjs, sparse, msk, samp_bias1, samp_bias2,
           W0, b0, a0, Wb0, bb0, W1, b1, a1, Wb1, bb1):
    x0 = _matmul(seq1, W0)
    x1 = _matmul(seq1, W1)
    x2 = _matmul(seq2, W0)
    x3 = _matmul(seq2, W1)

    src = adjs[:, 0, :].reshape(P, NT, EPT)
    dst = adjs[:, 1, :].reshape(P, NT, EPT)
    pad = ((0, 0), (0, 0), (0, PAD))
    srcp = jnp.pad(src, pad, constant_values=0).reshape(P, NT, NG, G, CH)
    dstp = jnp.pad(dst, pad, constant_values=N).reshape(P, NT, NG, G, CH)

    aggs = _segsum(x0, x1, x2, x3, srcp, dstp)

    bs = jnp.stack([b0, b1]).reshape(P, 1, H)
    alphas = jnp.stack([a0, a1]).reshape(P, 1, 1)
    msk3 = msk.reshape(NBLK, 1, NB)
    hh, h2, colsum, msksum = _activate(aggs, bs, alphas, msk3)

    wbs = jnp.stack([Wb0, Wb1])
    bbs = jnp.stack([bb0, bb1]).reshape(P, 1, 1)
    lbl3 = lbl.reshape(2 * NBLK, 1, NB)
    sb1_3 = samp_bias1.reshape(NBLK, 1, NB)
    sb2_3 = samp_bias2.reshape(NBLK, 1, NB)
    loss = _loss(hh, h2, colsum, msksum, wbs, bbs, lbl3, sb1_3, sb2_3)
    return (loss[0, 0], hh)


# speedup vs baseline: 3.5184x; 1.1390x over previous
import functools

import jax
import jax.numpy as jnp
from jax import lax
from jax.experimental import pallas as pl
from jax.experimental.pallas import tpu as pltpu
from jax.experimental.pallas import tpu_sc as plsc

N = 10000
E = 320000
F = 128
H = 128
P = 2

NT = 16
CH = 128
EPT = E // NT
G = 16
NG = -(-EPT // (G * CH))
EPT_PAD = NG * G * CH
PAD = EPT_PAD - EPT
ZR = 32
ZPT = 632
ACC_ROWS = NT * ZPT
XPT = 632
XPT_LAST = N - (NT - 1) * XPT

NB = 1000
NBLK = N // NB


def _mm_body(seq_ref, w_ref, out_ref):
    out_ref[...] = jnp.dot(seq_ref[0], w_ref[...],
                           preferred_element_type=jnp.float32)


def _matmul(seq, w):
    return pl.pallas_call(
        _mm_body,
        grid=(NBLK,),
        in_specs=[
            pl.BlockSpec((1, NB, F), lambda b: (0, b, 0)),
            pl.BlockSpec((F, H), lambda b: (0, 0)),
        ],
        out_specs=pl.BlockSpec((NB, H), lambda b: (b, 0)),
        out_shape=jax.ShapeDtypeStruct((N, H), jnp.float32),
    )(seq, w)


def _segsum_body(x0, x1, x2, x3, src_hbm, dst_hbm, out_hbm,
                 src_v, dst_v, rows_a, rows_b, zero_v, acc_sh, sem_a, sem_b):
    core = lax.axis_index("c")
    tid = lax.axis_index("s")
    xs = [x0, x1, x2, x3]

    def zrow(r, carry):
        for g in range(H // 16):
            zero_v[r, pl.ds(g * 16, 16)] = jnp.zeros((16,), jnp.float32)
        return carry
    lax.fori_loop(0, ZR, zrow, 0)

    for v in range(P):
        @pl.when(core == v)
        def _view():
            for rep in range(2):
                combo = rep * P + v
                x_hbm = xs[combo]
                zbase = pl.multiple_of(tid * ZPT, 8)
                for q in range(ZPT // ZR):
                    pltpu.sync_copy(zero_v,
                                    acc_sh.at[pl.ds(zbase + q * ZR, ZR)])
                rem = ZPT % ZR
                if rem:
                    pltpu.sync_copy(zero_v.at[pl.ds(0, rem)],
                                    acc_sh.at[pl.ds(zbase + (ZPT // ZR) * ZR, rem)])
                plsc.subcore_barrier()

                def group(g, carry):
                    pltpu.sync_copy(src_hbm.at[v, tid, g], src_v)
                    pltpu.sync_copy(dst_hbm.at[v, tid, g], dst_v)
                    bufs = (rows_a, rows_b)
                    sems = (sem_a, sem_b)
                    cps = [None, None]
                    cps[0] = pltpu.async_copy(x_hbm.at[src_v.at[0]],
                                              rows_a, sem_a)
                    for k in range(G):
                        p = k % 2
                        cps[p].wait()
                        if k + 1 < G:
                            q = (k + 1) % 2
                            cps[q] = pltpu.async_copy(
                                x_hbm.at[src_v.at[k + 1]], bufs[q], sems[q])
                        pltpu.sync_copy(bufs[p], acc_sh.at[dst_v.at[k]],
                                        add=True)
                    return carry
                lax.fori_loop(0, NG, group, 0)
                plsc.subcore_barrier()
                eb = pl.multiple_of(tid * XPT, 8)

                @pl.when(tid < NT - 1)
                def _exp():
                    pltpu.sync_copy(acc_sh.at[pl.ds(eb, XPT)],
                                    out_hbm.at[combo, pl.ds(eb, XPT)])

                @pl.when(tid == NT - 1)
                def _exp_last():
                    pltpu.sync_copy(acc_sh.at[pl.ds((NT - 1) * XPT, XPT_LAST)],
                                    out_hbm.at[combo, pl.ds((NT - 1) * XPT, XPT_LAST)])
                plsc.subcore_barrier()


@functools.cache
def _make_segsum():
    return pl.kernel(
        _segsum_body,
        out_type=jax.ShapeDtypeStruct((4, N, H), jnp.float32),
        mesh=plsc.VectorSubcoreMesh(core_axis_name="c", subcore_axis_name="s",
                                    num_cores=P, num_subcores=NT),
        scratch_types=[
            pltpu.VMEM((G, CH), jnp.int32),
            pltpu.VMEM((G, CH), jnp.int32),
            pltpu.VMEM((CH, H), jnp.float32),
            pltpu.VMEM((CH, H), jnp.float32),
            pltpu.VMEM((ZR, H), jnp.float32),
            pltpu.VMEM_SHARED((ACC_ROWS, H), jnp.float32),
            pltpu.SemaphoreType.DMA,
            pltpu.SemaphoreType.DMA,
        ],
    )


def _segsum(*args):
    return _make_segsum()(*args)


def _act_body(agg1_ref, agg2_ref, b_ref, a_ref, msk_ref,
              hh_ref, h2_ref, colsum_ref, msksum_ref):
    v = pl.program_id(0)
    b = pl.program_id(1)
    bias = b_ref[0, 0]
    alpha = a_ref[0, 0, 0]
    g1 = agg1_ref[0] + bias[None, :]
    h1 = jnp.where(g1 > 0, g1, alpha * g1)
    g2 = agg2_ref[0] + bias[None, :]
    h2 = jnp.where(g2 > 0, g2, alpha * g2)
    hh_ref[0, 0] = h1
    h2_ref[0] = h2
    m = msk_ref[0, 0]
    part = jnp.sum(h1 * m[:, None], axis=0)

    @pl.when(b == 0)
    def _init():
        colsum_ref[...] = part[None, None, :]

    @pl.when(b != 0)
    def _acc():
        colsum_ref[...] = colsum_ref[...] + part[None, None, :]

    @pl.when(jnp.logical_and(v == 0, b == 0))
    def _minit():
        msksum_ref[...] = jnp.sum(m).reshape(1, 1)

    @pl.when(jnp.logical_and(v == 0, b != 0))
    def _macc():
        msksum_ref[...] = msksum_ref[...] + jnp.sum(m).reshape(1, 1)


def _activate(aggs, bs, alphas, msk):
    return pl.pallas_call(
        _act_body,
        grid=(P, NBLK),
        in_specs=[
            pl.BlockSpec((1, NB, H), lambda v, b: (v, b, 0)),
            pl.BlockSpec((1, NB, H), lambda v, b: (v + P, b, 0)),
            pl.BlockSpec((1, 1, H), lambda v, b: (v, 0, 0)),
            pl.BlockSpec((1, 1, 1), lambda v, b: (v, 0, 0)),
            pl.BlockSpec((1, 1, NB), lambda v, b: (b, 0, 0)),
        ],
        out_specs=[
            pl.BlockSpec((1, 1, NB, H), lambda v, b: (v, 0, b, 0)),
            pl.BlockSpec((1, NB, H), lambda v, b: (v, b, 0)),
            pl.BlockSpec((1, 1, H), lambda v, b: (v, 0, 0)),
            pl.BlockSpec((1, 1), lambda v, b: (0, 0)),
        ],
        out_shape=[
            jax.ShapeDtypeStruct((P, 1, N, H), jnp.float32),
            jax.ShapeDtypeStruct((P, N, H), jnp.float32),
            jax.ShapeDtypeStruct((P, 1, H), jnp.float32),
            jax.ShapeDtypeStruct((1, 1), jnp.float32),
        ],
    )(aggs, aggs, bs, alphas, msk)


def _bce(x, y):
    return jnp.maximum(x, 0.0) - x * y + jnp.log(1.0 + jnp.exp(-jnp.abs(x)))


def _loss_body(hh_ref, h2_ref, colsum_ref, msksum_ref, wb_ref, bb_ref,
               lbl1_ref, lbl2_ref, sb1_ref, sb2_ref, loss_ref):
    v = pl.program_id(0)
    b = pl.program_id(1)
    c = colsum_ref[0, 0] / msksum_ref[0, 0]
    c = 1.0 / (1.0 + jnp.exp(-c))
    u = jnp.sum(wb_ref[0] * c[None, :], axis=1)
    bb = bb_ref[0, 0, 0]
    sc1 = jnp.sum(hh_ref[0, 0] * u[None, :], axis=1) + bb + sb1_ref[0, 0]
    sc2 = jnp.sum(h2_ref[0] * u[None, :], axis=1) + bb + sb2_ref[0, 0]
    part = (jnp.sum(_bce(sc1, lbl1_ref[0, 0]))
            + jnp.sum(_bce(sc2, lbl2_ref[0, 0])))

    @pl.when(jnp.logical_and(v == 0, b == 0))
    def _init():
        loss_ref[...] = jnp.zeros((1, 1), jnp.float32)

    loss_ref[...] = loss_ref[...] + part.reshape(1, 1)

    @pl.when(jnp.logical_and(v == P - 1, b == NBLK - 1))
    def _fin():
        loss_ref[...] = loss_ref[...] * (1.0 / (2.0 * N * P))


def _loss(hh, h2, colsum, msksum, wbs, bbs, lbl, sb1, sb2):
    return pl.pallas_call(
        _loss_body,
        grid=(P, NBLK),
        in_specs=[
            pl.BlockSpec((1, 1, NB, H), lambda v, b: (v, 0, b, 0)),
            pl.BlockSpec((1, NB, H), lambda v, b: (v, b, 0)),
            pl.BlockSpec((1, 1, H), lambda v, b: (v, 0, 0)),
            pl.BlockSpec((1, 1), lambda v, b: (0, 0)),
            pl.BlockSpec((1, H, H), lambda v, b: (v, 0, 0)),
            pl.BlockSpec((1, 1, 1), lambda v, b: (v, 0, 0)),
            pl.BlockSpec((1, 1, NB), lambda v, b: (b, 0, 0)),
            pl.BlockSpec((1, 1, NB), lambda v, b: (NBLK + b, 0, 0)),
            pl.BlockSpec((1, 1, NB), lambda v, b: (b, 0, 0)),
            pl.BlockSpec((1, 1, NB), lambda v, b: (b, 0, 0)),
        ],
        out_specs=pl.BlockSpec((1, 1), lambda v, b: (0, 0)),
        out_shape=jax.ShapeDtypeStruct((1, 1), jnp.float32),
    )(hh, h2, colsum, msksum, wbs, bbs, lbl, lbl, sb1, sb2)


def kernel(seq1, seq2, lbl, adjs, sparse, msk, samp_bias1, samp_bias2,
           W0, b0, a0, Wb0, bb0, W1, b1, a1, Wb1, bb1):
    x0 = _matmul(seq1, W0)
    x1 = _matmul(seq1, W1)
    x2 = _matmul(seq2, W0)
    x3 = _matmul(seq2, W1)

    src = adjs[:, 0, :].reshape(P, NT, EPT)
    dst = adjs[:, 1, :].reshape(P, NT, EPT)
    pad = ((0, 0), (0, 0), (0, PAD))
    srcp = jnp.pad(src, pad, constant_values=0).reshape(P, NT, NG, G, CH)
    dstp = jnp.pad(dst, pad, constant_values=N).reshape(P, NT, NG, G, CH)

    aggs = _segsum(x0, x1, x2, x3, srcp, dstp)

    bs = jnp.stack([b0, b1]).reshape(P, 1, H)
    alphas = jnp.stack([a0, a1]).reshape(P, 1, 1)
    msk3 = msk.reshape(NBLK, 1, NB)
    hh, h2, colsum, msksum = _activate(aggs, bs, alphas, msk3)

    wbs = jnp.stack([Wb0, Wb1])
    bbs = jnp.stack([bb0, bb1]).reshape(P, 1, 1)
    lbl3 = lbl.reshape(2 * NBLK, 1, NB)
    sb1_3 = samp_bias1.reshape(NBLK, 1, NB)
    sb2_3 = samp_bias2.reshape(NBLK, 1, NB)
    loss = _loss(hh, h2, colsum, msksum, wbs, bbs, lbl3, sb1_3, sb2_3)
    return (loss[0, 0], hh)
